# baseline (speedup 1.0000x reference)
import functools
import math

import jax
import jax.numpy as jnp
from jax.experimental import pallas as pl
from jax.experimental.pallas import tpu as pltpu

_BLK = 20000
_SPLIT = 2


def _gelu(v):
    return 0.5 * v * (1.0 + jax.lax.erf(v * (1.0 / math.sqrt(2.0))))


def _gelu2(v):
    return v * (1.0 + jax.lax.erf(v * (1.0 / math.sqrt(2.0))))


def _body(batch_ref, x_ref, u_ref,
          nw1_ref, nb1_ref, nw2_ref, nb2_ref,
          qw1_ref, qb1_ref, qw2_ref, qb2_ref,
          out_ref, uq_s, m_s, d_s, acc_s, gb_s, *, nblocks, nseg):
    i = pl.program_id(0)

    @pl.when(i == 0)
    def _init():
        uqh = _gelu(jnp.dot(u_ref[:], qw1_ref[:],
                            preferred_element_type=jnp.float32) + qb1_ref[:])
        uq = jnp.dot(uqh, qw2_ref[:],
                     preferred_element_type=jnp.float32) + qb2_ref[:]
        uq_sc = uq * (1.0 / math.sqrt(uq.shape[1]))
        uq_s[:] = uq_sc
        gb_s[:] = jax.lax.dot_general(
            uq_sc, nb2_ref[:], (((1,), (1,)), ((), ())),
            preferred_element_type=jnp.float32)
        m_s[:] = jnp.full(m_s.shape, -1e30, jnp.float32)
        d_s[:] = jnp.zeros(d_s.shape, jnp.float32)
        acc_s[:] = jnp.zeros(acc_s.shape, jnp.float32)

    sub = _BLK // _SPLIT
    iota_col = jax.lax.broadcasted_iota(jnp.int32, (nseg, 1), 0)
    xns, gate_owns, bmaxs = [], [], []
    for k in range(_SPLIT):
        x = x_ref[pl.ds(k * sub, sub), :].astype(jnp.bfloat16)
        h = _gelu2(jnp.dot(x, nw1_ref[:], preferred_element_type=jnp.float32)
                   + nb1_ref[:])
        xn0 = jnp.dot(h.astype(jnp.bfloat16), nw2_ref[:],
                      preferred_element_type=jnp.float32)
        gate_t = jax.lax.dot_general(
            uq_s[:], xn0, (((1,), (1,)), ((), ())),
            preferred_element_type=jnp.float32)
        seg = batch_ref[0, :, pl.ds(k * sub, sub)]
        gate_own = jnp.where(iota_col == seg, gate_t, -jnp.inf)
        xns.append(xn0)
        gate_owns.append(gate_own)
        bmaxs.append(jnp.max(gate_own, axis=1, keepdims=True))

    m_old = m_s[:]
    bm = bmaxs[0]
    for b in bmaxs[1:]:
        bm = jnp.maximum(bm, b)
    gb = gb_s[:]
    m_new = jnp.maximum(m_old, bm + gb)
    scale = jnp.exp(m_old - m_new)
    shift = m_new - gb

    d_blk = None
    mm = None
    for k in range(_SPLIT):
        p = jnp.exp(gate_owns[k] - shift)
        ds = jnp.sum(p, axis=1, keepdims=True)
        pa = jax.lax.dot_general(
            p, xns[k], (((1,), (0,)), ((), ())),
            preferred_element_type=jnp.float32)
        d_blk = ds if d_blk is None else d_blk + ds
        mm = pa if mm is None else mm + pa
    d_s[:] = d_s[:] * scale + d_blk
    acc_s[:] = acc_s[:] * scale + mm + d_blk * nb2_ref[:]
    m_s[:] = m_new

    @pl.when(i == nblocks - 1)
    def _fin():
        out_ref[:] = acc_s[:] / (d_s[:] + 1e-16)


def kernel(x, u, batch, size, node_w1, node_b1, node_w2, node_b2,
           ques_w1, ques_b1, ques_w2, ques_b2):
    n, d = x.shape
    nseg, c = u.shape
    nblocks = n // _BLK
    assert nblocks * _BLK == n

    batch3 = batch.reshape(nblocks, 1, _BLK)
    node_w1 = node_w1.astype(jnp.bfloat16)
    node_w2 = (node_w2 * 0.5).astype(jnp.bfloat16)
    nb1 = node_b1.reshape(1, c)
    nb2 = node_b2.reshape(1, c)
    qb1 = ques_b1.reshape(1, c)
    qb2 = ques_b2.reshape(1, c)

    full = lambda shape: pl.BlockSpec(shape, lambda i: (0,) * len(shape))
    out = pl.pallas_call(
        functools.partial(_body, nblocks=nblocks, nseg=nseg),
        grid=(nblocks,),
        in_specs=[
            pl.BlockSpec((1, 1, _BLK), lambda i: (i, 0, 0)),
            pl.BlockSpec((_BLK, d), lambda i: (i, 0)),
            full((nseg, c)),
            full((d, c)), full((1, c)), full((c, c)), full((1, c)),
            full((c, c)), full((1, c)), full((c, c)), full((1, c)),
        ],
        out_specs=pl.BlockSpec((nseg, c), lambda i: (0, 0)),
        out_shape=jax.ShapeDtypeStruct((nseg, c), jnp.float32),
        scratch_shapes=[
            pltpu.VMEM((nseg, c), jnp.float32),
            pltpu.VMEM((nseg, 1), jnp.float32),
            pltpu.VMEM((nseg, 1), jnp.float32),
            pltpu.VMEM((nseg, c), jnp.float32),
            pltpu.VMEM((nseg, 1), jnp.float32),
        ],
        compiler_params=pltpu.CompilerParams(
            dimension_semantics=("arbitrary",)),
    )(batch3, x, u, node_w1, nb1, node_w2, nb2, ques_w1, qb1, ques_w2, qb2)

    del size
    return out


# speedup vs baseline: 1.2946x; 1.2946x over previous
import functools
import math

import jax
import jax.numpy as jnp
from jax.experimental import pallas as pl
from jax.experimental.pallas import tpu as pltpu

_BLK = 20000
_SPLIT = 2


def _gelu(v):
    return 0.5 * v * (1.0 + jax.lax.erf(v * (1.0 / math.sqrt(2.0))))


def _gelu2(v):
    return v * (1.0 + jax.lax.erf(v * (1.0 / math.sqrt(2.0))))


def _body(batch_ref, x_ref, u_ref,
          nw1_ref, nb1_ref, nw2_ref, nb2_ref,
          qw1_ref, qb1_ref, qw2_ref, qb2_ref,
          out_ref, uq_s, m_s, d_s, acc_s, gb_s, *, nblocks, nseg):
    i = pl.program_id(0)

    @pl.when(i == 0)
    def _init():
        uqh = _gelu(jnp.dot(u_ref[:], qw1_ref[:],
                            preferred_element_type=jnp.float32) + qb1_ref[:])
        uq = jnp.dot(uqh, qw2_ref[:],
                     preferred_element_type=jnp.float32) + qb2_ref[:]
        uq_sc = uq * (1.0 / math.sqrt(uq.shape[1]))
        uq_s[:] = jax.lax.dot_general(
            uq_sc, nw2_ref[:], (((1,), (1,)), ((), ())),
            preferred_element_type=jnp.float32)
        gb_s[:] = jax.lax.dot_general(
            uq_sc, nb2_ref[:], (((1,), (1,)), ((), ())),
            preferred_element_type=jnp.float32)
        m_s[:] = jnp.full(m_s.shape, -1e30, jnp.float32)
        d_s[:] = jnp.zeros(d_s.shape, jnp.float32)
        acc_s[:] = jnp.zeros(acc_s.shape, jnp.float32)

    sub = _BLK // _SPLIT
    iota_col = jax.lax.broadcasted_iota(jnp.int32, (nseg, 1), 0)
    xns, gate_owns, bmaxs = [], [], []
    for k in range(_SPLIT):
        x = x_ref[pl.ds(k * sub, sub), :]
        h = _gelu2(jnp.dot(x, nw1_ref[:], preferred_element_type=jnp.float32)
                   + nb1_ref[:])
        xn0 = jnp.dot(h, nw2_ref[:], preferred_element_type=jnp.float32)
        gate_t = jax.lax.dot_general(
            uq_s[:], h, (((1,), (1,)), ((), ())),
            preferred_element_type=jnp.float32)
        seg = batch_ref[0, :, pl.ds(k * sub, sub)]
        gate_own = jnp.where(iota_col == seg, gate_t, -jnp.inf)
        xns.append(xn0)
        gate_owns.append(gate_own)
        bmaxs.append(jnp.max(gate_own, axis=1, keepdims=True))

    m_old = m_s[:]
    bm = bmaxs[0]
    for b in bmaxs[1:]:
        bm = jnp.maximum(bm, b)
    gb = gb_s[:]
    m_new = jnp.maximum(m_old, bm + gb)
    scale = jnp.exp(m_old - m_new)
    shift = m_new - gb

    d_blk = None
    mm = None
    for k in range(_SPLIT):
        p = jnp.exp(gate_owns[k] - shift)
        ds = jnp.sum(p, axis=1, keepdims=True)
        pa = jax.lax.dot_general(
            p, xns[k], (((1,), (0,)), ((), ())),
            preferred_element_type=jnp.float32)
        d_blk = ds if d_blk is None else d_blk + ds
        mm = pa if mm is None else mm + pa
    d_s[:] = d_s[:] * scale + d_blk
    acc_s[:] = acc_s[:] * scale + mm + d_blk * nb2_ref[:]
    m_s[:] = m_new

    @pl.when(i == nblocks - 1)
    def _fin():
        out_ref[:] = acc_s[:] / (d_s[:] + 1e-16)


def kernel(x, u, batch, size, node_w1, node_b1, node_w2, node_b2,
           ques_w1, ques_b1, ques_w2, ques_b2):
    n, d = x.shape
    nseg, c = u.shape
    nblocks = n // _BLK
    assert nblocks * _BLK == n

    batch3 = batch.reshape(nblocks, 1, _BLK)
    node_w2 = node_w2 * 0.5
    nb1 = node_b1.reshape(1, c)
    nb2 = node_b2.reshape(1, c)
    qb1 = ques_b1.reshape(1, c)
    qb2 = ques_b2.reshape(1, c)

    full = lambda shape: pl.BlockSpec(shape, lambda i: (0,) * len(shape))
    out = pl.pallas_call(
        functools.partial(_body, nblocks=nblocks, nseg=nseg),
        grid=(nblocks,),
        in_specs=[
            pl.BlockSpec((1, 1, _BLK), lambda i: (i, 0, 0)),
            pl.BlockSpec((_BLK, d), lambda i: (i, 0)),
            full((nseg, c)),
            full((d, c)), full((1, c)), full((c, c)), full((1, c)),
            full((c, c)), full((1, c)), full((c, c)), full((1, c)),
        ],
        out_specs=pl.BlockSpec((nseg, c), lambda i: (0, 0)),
        out_shape=jax.ShapeDtypeStruct((nseg, c), jnp.float32),
        scratch_shapes=[
            pltpu.VMEM((nseg, c), jnp.float32),
            pltpu.VMEM((nseg, 1), jnp.float32),
            pltpu.VMEM((nseg, 1), jnp.float32),
            pltpu.VMEM((nseg, c), jnp.float32),
            pltpu.VMEM((nseg, 1), jnp.float32),
        ],
        compiler_params=pltpu.CompilerParams(
            dimension_semantics=("arbitrary",)),
    )(batch3, x, u, node_w1, nb1, node_w2, nb2, ques_w1, qb1, ques_w2, qb2)

    del size
    return out


# speedup vs baseline: 1.4028x; 1.0836x over previous
import functools
import math

import jax
import jax.numpy as jnp
from jax.experimental import pallas as pl
from jax.experimental.pallas import tpu as pltpu

_BLK = 20000
_SPLIT = 2


def _gelu(v):
    return 0.5 * v * (1.0 + jax.lax.erf(v * (1.0 / math.sqrt(2.0))))


def _gelu2(v):
    return v * (1.0 + jax.lax.erf(v * (1.0 / math.sqrt(2.0))))


def _body(batch_ref, x_ref, u_ref,
          nw1_ref, nb1_ref, nw2_ref, nb2_ref,
          qw1_ref, qb1_ref, qw2_ref, qb2_ref,
          out_ref, uq_s, m_s, d_s, acc_s, gb_s, *, nblocks, nseg):
    i = pl.program_id(0)

    @pl.when(i == 0)
    def _init():
        uqh = _gelu(jnp.dot(u_ref[:], qw1_ref[:],
                            preferred_element_type=jnp.float32) + qb1_ref[:])
        uq = jnp.dot(uqh, qw2_ref[:],
                     preferred_element_type=jnp.float32) + qb2_ref[:]
        uq_sc = uq * (0.5 / math.sqrt(uq.shape[1]))
        uq_s[:] = uq_sc
        gb_s[:] = jax.lax.dot_general(
            uq * (1.0 / math.sqrt(uq.shape[1])), nb2_ref[:],
            (((1,), (1,)), ((), ())),
            preferred_element_type=jnp.float32)
        m_s[:] = jnp.full(m_s.shape, -1e30, jnp.float32)
        d_s[:] = jnp.zeros(d_s.shape, jnp.float32)
        acc_s[:] = jnp.zeros(acc_s.shape, jnp.float32)

    sub = _BLK // _SPLIT
    iota_col = jax.lax.broadcasted_iota(jnp.int32, (nseg, 1), 0)
    xns, gate_owns, bmaxs = [], [], []
    for k in range(_SPLIT):
        x = x_ref[pl.ds(k * sub, sub), :]
        h = _gelu2(jnp.dot(x, nw1_ref[:], preferred_element_type=jnp.float32)
                   + nb1_ref[:])
        xn0 = jnp.dot(h, nw2_ref[:], preferred_element_type=jnp.float32)
        gate_t = jax.lax.dot_general(
            uq_s[:], xn0, (((1,), (1,)), ((), ())),
            preferred_element_type=jnp.float32)
        seg = batch_ref[0, :, pl.ds(k * sub, sub)]
        gate_own = jnp.where(iota_col == seg, gate_t, -jnp.inf)
        xns.append(xn0)
        gate_owns.append(gate_own)
        bmaxs.append(jnp.max(gate_own, axis=1, keepdims=True))

    m_old = m_s[:]
    bm = bmaxs[0]
    for b in bmaxs[1:]:
        bm = jnp.maximum(bm, b)
    gb = gb_s[:]
    m_new = jnp.maximum(m_old, bm + gb)
    scale = jnp.exp(m_old - m_new)
    shift = m_new - gb

    d_blk = None
    mm = None
    for k in range(_SPLIT):
        p = jnp.exp(gate_owns[k] - shift)
        ds = jnp.sum(p, axis=1, keepdims=True)
        pa = jax.lax.dot_general(
            p, xns[k], (((1,), (0,)), ((), ())),
            preferred_element_type=jnp.float32)
        d_blk = ds if d_blk is None else d_blk + ds
        mm = pa if mm is None else mm + pa
    d_s[:] = d_s[:] * scale + d_blk
    acc_s[:] = acc_s[:] * scale + mm
    m_s[:] = m_new

    @pl.when(i == nblocks - 1)
    def _fin():
        inv = 1.0 / (d_s[:] + 1e-16)
        out_ref[:] = (0.5 * inv) * acc_s[:] + (d_s[:] * inv) * nb2_ref[:]


def kernel(x, u, batch, size, node_w1, node_b1, node_w2, node_b2,
           ques_w1, ques_b1, ques_w2, ques_b2):
    n, d = x.shape
    nseg, c = u.shape
    nblocks = n // _BLK
    assert nblocks * _BLK == n

    batch3 = batch.reshape(nblocks, 1, _BLK)
    nb1 = node_b1.reshape(1, c)
    nb2 = node_b2.reshape(1, c)
    qb1 = ques_b1.reshape(1, c)
    qb2 = ques_b2.reshape(1, c)

    full = lambda shape: pl.BlockSpec(shape, lambda i: (0,) * len(shape))
    out = pl.pallas_call(
        functools.partial(_body, nblocks=nblocks, nseg=nseg),
        grid=(nblocks,),
        in_specs=[
            pl.BlockSpec((1, 1, _BLK), lambda i: (i, 0, 0)),
            pl.BlockSpec((_BLK, d), lambda i: (i, 0)),
            full((nseg, c)),
            full((d, c)), full((1, c)), full((c, c)), full((1, c)),
            full((c, c)), full((1, c)), full((c, c)), full((1, c)),
        ],
        out_specs=pl.BlockSpec((nseg, c), lambda i: (0, 0)),
        out_shape=jax.ShapeDtypeStruct((nseg, c), jnp.float32),
        scratch_shapes=[
            pltpu.VMEM((nseg, c), jnp.float32),
            pltpu.VMEM((nseg, 1), jnp.float32),
            pltpu.VMEM((nseg, 1), jnp.float32),
            pltpu.VMEM((nseg, c), jnp.float32),
            pltpu.VMEM((nseg, 1), jnp.float32),
        ],
        compiler_params=pltpu.CompilerParams(
            dimension_semantics=("arbitrary",)),
    )(batch3, x, u, node_w1, nb1, node_w2, nb2, ques_w1, qb1, ques_w2, qb2)

    del size
    return out
